# baseline (speedup 1.0000x reference)
import functools

import jax
import jax.numpy as jnp
from jax import lax
from jax.experimental import pallas as pl
from jax.experimental.pallas import tpu as pltpu
from jax.experimental.pallas import tpu_sc as plsc

N = 10000
E = 320000
D = 128
H = 64
C = 10
G = 64

NC = 2
NS = 16
NW = NC * NS
CHUNK = 128
CH = 80
E_PAD = NW * CH * CHUNK
N2 = 10240
BLK = 1024
NB = N2 // BLK


def _deg_body(row_hbm, ones_hbm, zeros_hbm, out_hbm, rowv, ones_v, dacc,
              s0, s1, s2, s3):
    ssems = (s0, s1, s2, s3)
    nsq = len(ssems)
    c = lax.axis_index("c")
    s = lax.axis_index("s")
    wid = s * NC + c
    pltpu.sync_copy(row_hbm.at[wid], rowv)
    pltpu.sync_copy(ones_hbm, ones_v)
    z = N2 // NS
    pltpu.sync_copy(zeros_hbm.at[pl.ds(s * z, z)], dacc.at[pl.ds(s * z, z)])
    plsc.subcore_barrier()

    for b in range(nsq):
        pltpu.async_copy(ones_v, dacc.at[rowv.at[b]], ssems[b], add=True)

    def body(t, carry):
        for b in range(nsq):
            j = nsq * t + b
            j2 = j + nsq
            pltpu.make_async_copy(ones_v, dacc.at[rowv.at[j]],
                                  ssems[b]).wait()

            @pl.when(j2 < CH)
            def _():
                pltpu.async_copy(ones_v, dacc.at[rowv.at[j2]], ssems[b],
                                 add=True)

        return carry

    lax.fori_loop(0, CH // nsq, body, 0)
    plsc.subcore_barrier()
    pltpu.sync_copy(dacc.at[pl.ds(s * z, z)], out_hbm.at[c, pl.ds(s * z, z)])


@functools.cache
def _deg_call_cached():
    return pl.kernel(
        _deg_body,
        out_type=jax.ShapeDtypeStruct((NC, N2, 16), jnp.float32),
        mesh=plsc.VectorSubcoreMesh(core_axis_name="c", subcore_axis_name="s"),
        compiler_params=pltpu.CompilerParams(use_tc_tiling_on_sc=False),
        scratch_types=[
            pltpu.VMEM((CH, CHUNK), jnp.int32),
            pltpu.VMEM((CHUNK, 16), jnp.float32),
            pltpu.VMEM_SHARED((N2, 16), jnp.float32),
            pltpu.SemaphoreType.DMA,
            pltpu.SemaphoreType.DMA,
            pltpu.SemaphoreType.DMA,
            pltpu.SemaphoreType.DMA,
        ],
    )


def _deg_call(*args):
    return _deg_call_cached()(*args)


NBUF = 2


def _prop_body(src_hbm, col_hbm, row_hbm, zeros_hbm, out_hbm,
               colv, rowv, acc, src_sh, *bufs_and_sems):
    bufs = bufs_and_sems[:NBUF]
    gsems = bufs_and_sems[NBUF:2 * NBUF]
    ssems = bufs_and_sems[2 * NBUF:3 * NBUF]
    c = lax.axis_index("c")
    s = lax.axis_index("s")
    wid = s * NC + c
    pltpu.sync_copy(col_hbm.at[wid], colv)
    pltpu.sync_copy(row_hbm.at[wid], rowv)
    z = N2 // NS
    pltpu.sync_copy(zeros_hbm.at[pl.ds(s * z, z)], acc.at[pl.ds(s * z, z)])
    pltpu.sync_copy(src_hbm.at[pl.ds(s * z, z)], src_sh.at[pl.ds(s * z, z)])
    plsc.subcore_barrier()

    for b in range(NBUF):
        pltpu.async_copy(src_sh.at[colv.at[b]], bufs[b], gsems[b])

    def body(t, carry):
        for b in range(NBUF):
            j = NBUF * t + b
            pltpu.make_async_copy(src_sh.at[colv.at[j]], bufs[b],
                                  gsems[b]).wait()
            pltpu.async_copy(bufs[b], acc.at[rowv.at[j]], ssems[b], add=True)
        for b in range(NBUF):
            j = NBUF * t + b
            j2 = j + NBUF
            pltpu.make_async_copy(bufs[b], acc.at[rowv.at[j]],
                                  ssems[b]).wait()

            @pl.when(j2 < CH)
            def _():
                pltpu.async_copy(src_sh.at[colv.at[j2]], bufs[b], gsems[b])

        return carry

    lax.fori_loop(0, CH // NBUF, body, 0)
    plsc.subcore_barrier()
    pltpu.sync_copy(acc.at[pl.ds(s * z, z)], out_hbm.at[c, pl.ds(s * z, z)])


@functools.cache
def _prop_call_cached():
    return pl.kernel(
        _prop_body,
        out_type=jax.ShapeDtypeStruct((NC, N2, H), jnp.float32),
        mesh=plsc.VectorSubcoreMesh(core_axis_name="c", subcore_axis_name="s"),
        compiler_params=pltpu.CompilerParams(use_tc_tiling_on_sc=False),
        scratch_types=(
            [
                pltpu.VMEM((CH, CHUNK), jnp.int32),
                pltpu.VMEM((CH, CHUNK), jnp.int32),
                pltpu.VMEM_SHARED((N2, H), jnp.float32),
                pltpu.VMEM_SHARED((N2, H), jnp.float32),
            ]
            + [pltpu.VMEM((CHUNK, H), jnp.float32) for _ in range(NBUF)]
            + [pltpu.SemaphoreType.DMA for _ in range(2 * NBUF)]
        ),
    )


def _prop_call(*args):
    return _prop_call_cached()(*args)


def _mm0_body(deg_ref, x_ref, w_ref, b_ref, dinv_ref, out0_ref, src_ref):
    deg = deg_ref[0, :, 0:1] + deg_ref[1, :, 0:1]
    dinv = jnp.where(deg > 0, lax.rsqrt(deg), 0.0)
    d64 = jnp.broadcast_to(dinv, (BLK, H))
    o = jnp.dot(x_ref[...], w_ref[...], preferred_element_type=jnp.float32)
    o = o + b_ref[...]
    dinv_ref[...] = d64
    out0_ref[...] = o
    src_ref[...] = d64 * o


_mm0_call = pl.pallas_call(
    _mm0_body,
    grid=(NB,),
    in_specs=[
        pl.BlockSpec((NC, BLK, 16), lambda i: (0, i, 0)),
        pl.BlockSpec((BLK, D), lambda i: (i, 0)),
        pl.BlockSpec((D, H), lambda i: (0, 0)),
        pl.BlockSpec((1, H), lambda i: (0, 0)),
    ],
    out_specs=[
        pl.BlockSpec((BLK, H), lambda i: (i, 0)),
        pl.BlockSpec((BLK, H), lambda i: (i, 0)),
        pl.BlockSpec((BLK, H), lambda i: (i, 0)),
    ],
    out_shape=[
        jax.ShapeDtypeStruct((N2, H), jnp.float32),
        jax.ShapeDtypeStruct((N2, H), jnp.float32),
        jax.ShapeDtypeStruct((N2, H), jnp.float32),
    ],
)


def _ew_b_body(out0_ref, sp_ref, dinv_ref, a_ref, out1_ref, src_ref):
    d = dinv_ref[...]
    svm = sp_ref[0] + sp_ref[1]
    o1 = out0_ref[...] + a_ref[0, 0] * d * svm
    out1_ref[...] = o1
    src_ref[...] = d * o1


_ew_b_call = pl.pallas_call(
    _ew_b_body,
    grid=(NB,),
    in_specs=[
        pl.BlockSpec((BLK, H), lambda i: (i, 0)),
        pl.BlockSpec((NC, BLK, H), lambda i: (0, i, 0)),
        pl.BlockSpec((BLK, H), lambda i: (i, 0)),
        pl.BlockSpec(memory_space=pltpu.SMEM),
    ],
    out_specs=[
        pl.BlockSpec((BLK, H), lambda i: (i, 0)),
        pl.BlockSpec((BLK, H), lambda i: (i, 0)),
    ],
    out_shape=[
        jax.ShapeDtypeStruct((N2, H), jnp.float32),
        jax.ShapeDtypeStruct((N2, H), jnp.float32),
    ],
)


def _ew_c_body(sp_ref, dinv_ref, src_ref):
    d = dinv_ref[...]
    src_ref[...] = d * d * (sp_ref[0] + sp_ref[1])


_ew_c_call = pl.pallas_call(
    _ew_c_body,
    grid=(NB,),
    in_specs=[
        pl.BlockSpec((NC, BLK, H), lambda i: (0, i, 0)),
        pl.BlockSpec((BLK, H), lambda i: (i, 0)),
    ],
    out_specs=pl.BlockSpec((BLK, H), lambda i: (i, 0)),
    out_shape=jax.ShapeDtypeStruct((N2, H), jnp.float32),
)


def _relumm_body(out1_ref, sp_ref, dinv_ref, a_ref, w_ref, b_ref,
                 out0_ref, src_ref):
    d = dinv_ref[...]
    h = jnp.maximum(out1_ref[...] + a_ref[0, 0] * d * (sp_ref[0] + sp_ref[1]),
                    0.0)
    o = jnp.dot(h, w_ref[...], preferred_element_type=jnp.float32)
    o = o + b_ref[...]
    d64 = d
    out0_ref[...] = o
    src_ref[...] = d64 * o


_relumm_call = pl.pallas_call(
    _relumm_body,
    grid=(NB,),
    in_specs=[
        pl.BlockSpec((BLK, H), lambda i: (i, 0)),
        pl.BlockSpec((NC, BLK, H), lambda i: (0, i, 0)),
        pl.BlockSpec((BLK, H), lambda i: (i, 0)),
        pl.BlockSpec(memory_space=pltpu.SMEM),
        pl.BlockSpec((H, H), lambda i: (0, 0)),
        pl.BlockSpec((1, H), lambda i: (0, 0)),
    ],
    out_specs=[
        pl.BlockSpec((BLK, H), lambda i: (i, 0)),
        pl.BlockSpec((BLK, H), lambda i: (i, 0)),
    ],
    out_shape=[
        jax.ShapeDtypeStruct((N2, H), jnp.float32),
        jax.ShapeDtypeStruct((N2, H), jnp.float32),
    ],
)


def _final_body(out1_ref, sp_ref, dinv_ref, a_ref, bt_ref, wc_ref, bc_ref,
                out_ref, sums, counts):
    i = pl.program_id(0)

    @pl.when(i == 0)
    def _():
        sums[...] = jnp.zeros_like(sums)
        counts[...] = jnp.zeros_like(counts)

    d = dinv_ref[...]
    h = jnp.maximum(out1_ref[...] + a_ref[0, 0] * d * (sp_ref[0] + sp_ref[1]),
                    0.0)
    bt = bt_ref[0, 0]
    gids = lax.broadcasted_iota(jnp.int32, (G, BLK), 0)
    p = (bt[None, :] == gids).astype(jnp.float32)
    sums[...] += jnp.dot(p, h, preferred_element_type=jnp.float32)
    counts[...] += jnp.dot(p, jnp.ones((BLK, H), jnp.float32),
                           preferred_element_type=jnp.float32)

    @pl.when(i == NB - 1)
    def _():
        pooled = sums[...] / jnp.maximum(counts[...], 1.0)
        out_ref[...] = jnp.dot(pooled, wc_ref[...],
                               preferred_element_type=jnp.float32) + bc_ref[...]


_final_call = pl.pallas_call(
    _final_body,
    grid=(NB,),
    in_specs=[
        pl.BlockSpec((BLK, H), lambda i: (i, 0)),
        pl.BlockSpec((NC, BLK, H), lambda i: (0, i, 0)),
        pl.BlockSpec((BLK, H), lambda i: (i, 0)),
        pl.BlockSpec(memory_space=pltpu.SMEM),
        pl.BlockSpec((1, 1, BLK), lambda i: (i, 0, 0)),
        pl.BlockSpec((H, C), lambda i: (0, 0)),
        pl.BlockSpec((1, C), lambda i: (0, 0)),
    ],
    out_specs=pl.BlockSpec((G, C), lambda i: (0, 0)),
    out_shape=jax.ShapeDtypeStruct((G, C), jnp.float32),
    scratch_shapes=[
        pltpu.VMEM((G, H), jnp.float32),
        pltpu.VMEM((G, H), jnp.float32),
    ],
)


def kernel(x, edge_index, batch, W0, b0, a0, W1, b1, a1, W2, b2, a2, Wc, bc):
    row = edge_index[0]
    col = edge_index[1]
    pad = E_PAD - E
    rowp = jnp.concatenate(
        [row, jnp.full((pad,), N, jnp.int32)]).reshape(NW, CH, CHUNK)
    colp = jnp.concatenate(
        [col, jnp.full((pad,), N, jnp.int32)]).reshape(NW, CH, CHUNK)
    xp = jnp.pad(x, ((0, N2 - N), (0, 0)))
    batch2 = jnp.concatenate(
        [batch, jnp.full((N2 - N,), G, jnp.int32)]).reshape(NB, 1, BLK)
    zeros64 = jnp.zeros((N2, H), jnp.float32)
    zeros16 = jnp.zeros((N2, 16), jnp.float32)
    ones16 = jnp.ones((CHUNK, 16), jnp.float32)

    deg_part = _deg_call(rowp, ones16, zeros16)
    dinv64, out0, src = _mm0_call(deg_part, xp, W0.T, b0.reshape(1, H))

    out = None
    for (W, b, a) in ((W0, b0, a0), (W1, b1, a1), (W2, b2, a2)):
        a0s = a[0:1].reshape(1, 1)
        a1s = a[1:2].reshape(1, 1)
        sp = _prop_call(src, colp, rowp, zeros64)
        out1, src = _ew_b_call(out0, sp, dinv64, a0s)
        sp = _prop_call(src, colp, rowp, zeros64)
        src = _ew_c_call(sp, dinv64)
        sp = _prop_call(src, colp, rowp, zeros64)
        if W is W2:
            out = _final_call(out1, sp, dinv64, a1s, batch2, Wc.T,
                              bc.reshape(1, C))
        elif W is W0:
            out0, src = _relumm_call(out1, sp, dinv64, a1s, W1.T,
                                     b1.reshape(1, H))
        else:
            out0, src = _relumm_call(out1, sp, dinv64, a1s, W2.T,
                                     b2.reshape(1, H))
    return out


# speedup vs baseline: 1.0373x; 1.0373x over previous
import functools

import jax
import jax.numpy as jnp
from jax import lax
from jax.experimental import pallas as pl
from jax.experimental.pallas import tpu as pltpu
from jax.experimental.pallas import tpu_sc as plsc

N = 10000
E = 320000
D = 128
H = 64
C = 10
G = 64

NC = 2
NS = 16
NW = NC * NS
CHUNK = 128
CH = 80
E_PAD = NW * CH * CHUNK
N2 = 10240
BLK = 1024
NB = N2 // BLK


def _deg_body(row_hbm, ones_hbm, zeros_hbm, out_hbm, rowv, ones_v, dacc,
              s0, s1, s2, s3):
    ssems = (s0, s1, s2, s3)
    nsq = len(ssems)
    c = lax.axis_index("c")
    s = lax.axis_index("s")
    wid = s * NC + c
    pltpu.sync_copy(row_hbm.at[wid], rowv)
    pltpu.sync_copy(ones_hbm, ones_v)
    z = N2 // NS
    pltpu.sync_copy(zeros_hbm.at[pl.ds(s * z, z)], dacc.at[pl.ds(s * z, z)])
    plsc.subcore_barrier()

    for b in range(nsq):
        pltpu.async_copy(ones_v, dacc.at[rowv.at[b]], ssems[b], add=True)

    def body(t, carry):
        for b in range(nsq):
            j = nsq * t + b
            j2 = j + nsq
            pltpu.make_async_copy(ones_v, dacc.at[rowv.at[j]],
                                  ssems[b]).wait()

            @pl.when(j2 < CH)
            def _():
                pltpu.async_copy(ones_v, dacc.at[rowv.at[j2]], ssems[b],
                                 add=True)

        return carry

    lax.fori_loop(0, CH // nsq, body, 0)
    plsc.subcore_barrier()
    pltpu.sync_copy(dacc.at[pl.ds(s * z, z)], out_hbm.at[c, pl.ds(s * z, z)])


@functools.cache
def _deg_call_cached():
    return pl.kernel(
        _deg_body,
        out_type=jax.ShapeDtypeStruct((NC, N2, 16), jnp.float32),
        mesh=plsc.VectorSubcoreMesh(core_axis_name="c", subcore_axis_name="s"),
        compiler_params=pltpu.CompilerParams(use_tc_tiling_on_sc=False),
        scratch_types=[
            pltpu.VMEM((CH, CHUNK), jnp.int32),
            pltpu.VMEM((CHUNK, 16), jnp.float32),
            pltpu.VMEM_SHARED((N2, 16), jnp.float32),
            pltpu.SemaphoreType.DMA,
            pltpu.SemaphoreType.DMA,
            pltpu.SemaphoreType.DMA,
            pltpu.SemaphoreType.DMA,
        ],
    )


def _deg_call(*args):
    return _deg_call_cached()(*args)


NBUF = 2


def _prop_body(src_hbm, col_hbm, row_hbm, zeros_hbm, out_hbm,
               colv, rowv, acc, src_sh, *bufs_and_sems):
    bufs = bufs_and_sems[:NBUF]
    gsems = bufs_and_sems[NBUF:2 * NBUF]
    ssems = bufs_and_sems[2 * NBUF:3 * NBUF]
    c = lax.axis_index("c")
    s = lax.axis_index("s")
    wid = s * NC + c
    pltpu.sync_copy(col_hbm.at[wid], colv)
    pltpu.sync_copy(row_hbm.at[wid], rowv)
    z = N2 // NS
    pltpu.sync_copy(zeros_hbm.at[pl.ds(s * z, z)], acc.at[pl.ds(s * z, z)])
    pltpu.sync_copy(src_hbm.at[pl.ds(s * z, z)], src_sh.at[pl.ds(s * z, z)])
    plsc.subcore_barrier()

    for b in range(NBUF):
        pltpu.async_copy(src_sh.at[colv.at[b]], bufs[b], gsems[b])

    def body(t, carry):
        for b in range(NBUF):
            j = NBUF * t + b
            pltpu.make_async_copy(src_sh.at[colv.at[j]], bufs[b],
                                  gsems[b]).wait()
            pltpu.async_copy(bufs[b], acc.at[rowv.at[j]], ssems[b], add=True)
        for b in range(NBUF):
            j = NBUF * t + b
            j2 = j + NBUF
            pltpu.make_async_copy(bufs[b], acc.at[rowv.at[j]],
                                  ssems[b]).wait()

            @pl.when(j2 < CH)
            def _():
                pltpu.async_copy(src_sh.at[colv.at[j2]], bufs[b], gsems[b])

        return carry

    lax.fori_loop(0, CH // NBUF, body, 0)
    plsc.subcore_barrier()
    pltpu.sync_copy(acc.at[pl.ds(s * z, z)], out_hbm.at[c, pl.ds(s * z, z)])


@functools.cache
def _prop_call_cached():
    return pl.kernel(
        _prop_body,
        out_type=jax.ShapeDtypeStruct((NC, N2, H), jnp.float32),
        mesh=plsc.VectorSubcoreMesh(core_axis_name="c", subcore_axis_name="s"),
        compiler_params=pltpu.CompilerParams(use_tc_tiling_on_sc=False),
        scratch_types=(
            [
                pltpu.VMEM((CH, CHUNK), jnp.int32),
                pltpu.VMEM((CH, CHUNK), jnp.int32),
                pltpu.VMEM_SHARED((N2, H), jnp.float32),
                pltpu.VMEM_SHARED((N2, H), jnp.float32),
            ]
            + [pltpu.VMEM((CHUNK, H), jnp.float32) for _ in range(NBUF)]
            + [pltpu.SemaphoreType.DMA for _ in range(2 * NBUF)]
        ),
    )


def _prop_call(*args):
    return _prop_call_cached()(*args)


RW = 32
NQ = (N2 // NS) // RW


def _prop3_body(spart_hbm, dinv_hbm, col_hbm, row_hbm, zeros_hbm, out_hbm,
                colv, rowv, acc, src_sh, *bufs_and_sems):
    bufs = bufs_and_sems[:NBUF]
    gsems = bufs_and_sems[NBUF:2 * NBUF]
    ssems = bufs_and_sems[2 * NBUF:3 * NBUF]
    osems = bufs_and_sems[3 * NBUF:4 * NBUF]
    c = lax.axis_index("c")
    s = lax.axis_index("s")
    wid = s * NC + c
    pltpu.sync_copy(col_hbm.at[wid], colv)
    pltpu.sync_copy(row_hbm.at[wid], rowv)
    z = N2 // NS
    base_row = s * z
    pltpu.sync_copy(zeros_hbm.at[pl.ds(base_row, z)],
                    acc.at[pl.ds(base_row, z)])

    def _issue_in(q, b):
        r0 = base_row + q * RW
        pltpu.async_copy(dinv_hbm.at[pl.ds(r0, RW)],
                         bufs[b].at[pl.ds(0, RW)], gsems[b])
        pltpu.async_copy(spart_hbm.at[0, pl.ds(r0, RW)],
                         bufs[b].at[pl.ds(RW, RW)], gsems[b])
        pltpu.async_copy(spart_hbm.at[1, pl.ds(r0, RW)],
                         bufs[b].at[pl.ds(2 * RW, RW)], gsems[b])

    def _wait_in(q, b):
        r0 = base_row + q * RW
        pltpu.make_async_copy(dinv_hbm.at[pl.ds(r0, RW)],
                              bufs[b].at[pl.ds(0, RW)], gsems[b]).wait()
        pltpu.make_async_copy(spart_hbm.at[0, pl.ds(r0, RW)],
                              bufs[b].at[pl.ds(RW, RW)], gsems[b]).wait()
        pltpu.make_async_copy(spart_hbm.at[1, pl.ds(r0, RW)],
                              bufs[b].at[pl.ds(2 * RW, RW)], gsems[b]).wait()

    for b in range(NBUF):
        _issue_in(b, b)

    def ew_round(t, carry):
        for b in range(NBUF):
            q = NBUF * t + b
            _wait_in(q, b)

            @pl.when(t > 0)
            def _():
                pltpu.make_async_copy(
                    bufs[b].at[pl.ds(3 * RW, RW)],
                    src_sh.at[pl.ds(base_row, RW)], osems[b]).wait()

            def ew_row(r, carry2):
                for col in range(H // 16):
                    sl = pl.ds(col * 16, 16)
                    d = bufs[b][r, sl]
                    v = bufs[b][RW + r, sl] + bufs[b][2 * RW + r, sl]
                    bufs[b][3 * RW + r, sl] = d * d * v
                return carry2

            lax.fori_loop(0, RW, ew_row, 0)
            pltpu.async_copy(bufs[b].at[pl.ds(3 * RW, RW)],
                             src_sh.at[pl.ds(base_row + q * RW, RW)],
                             osems[b])

            @pl.when(q + NBUF < NQ)
            def _():
                _issue_in(q + NBUF, b)

        return carry

    lax.fori_loop(0, NQ // NBUF, ew_round, 0)
    for b in range(NBUF):
        pltpu.make_async_copy(bufs[b].at[pl.ds(3 * RW, RW)],
                              src_sh.at[pl.ds(base_row, RW)], osems[b]).wait()
    plsc.subcore_barrier()

    for b in range(NBUF):
        pltpu.async_copy(src_sh.at[colv.at[b]], bufs[b], gsems[b])

    def body(t, carry):
        for b in range(NBUF):
            j = NBUF * t + b
            pltpu.make_async_copy(src_sh.at[colv.at[j]], bufs[b],
                                  gsems[b]).wait()
            pltpu.async_copy(bufs[b], acc.at[rowv.at[j]], ssems[b], add=True)
        for b in range(NBUF):
            j = NBUF * t + b
            j2 = j + NBUF
            pltpu.make_async_copy(bufs[b], acc.at[rowv.at[j]],
                                  ssems[b]).wait()

            @pl.when(j2 < CH)
            def _():
                pltpu.async_copy(src_sh.at[colv.at[j2]], bufs[b], gsems[b])

        return carry

    lax.fori_loop(0, CH // NBUF, body, 0)
    plsc.subcore_barrier()
    pltpu.sync_copy(acc.at[pl.ds(base_row, z)], out_hbm.at[c, pl.ds(base_row, z)])


@functools.cache
def _prop3_call_cached():
    return pl.kernel(
        _prop3_body,
        out_type=jax.ShapeDtypeStruct((NC, N2, H), jnp.float32),
        mesh=plsc.VectorSubcoreMesh(core_axis_name="c", subcore_axis_name="s"),
        compiler_params=pltpu.CompilerParams(use_tc_tiling_on_sc=False),
        scratch_types=(
            [
                pltpu.VMEM((CH, CHUNK), jnp.int32),
                pltpu.VMEM((CH, CHUNK), jnp.int32),
                pltpu.VMEM_SHARED((N2, H), jnp.float32),
                pltpu.VMEM_SHARED((N2, H), jnp.float32),
            ]
            + [pltpu.VMEM((CHUNK, H), jnp.float32) for _ in range(NBUF)]
            + [pltpu.SemaphoreType.DMA for _ in range(3 * NBUF)]
        ),
    )


def _prop3_call(*args):
    return _prop3_call_cached()(*args)


def _mm0_body(deg_ref, x_ref, w_ref, b_ref, dinv_ref, out0_ref, src_ref):
    deg = deg_ref[0, :, 0:1] + deg_ref[1, :, 0:1]
    dinv = jnp.where(deg > 0, lax.rsqrt(deg), 0.0)
    d64 = jnp.broadcast_to(dinv, (BLK, H))
    o = jnp.dot(x_ref[...], w_ref[...], preferred_element_type=jnp.float32)
    o = o + b_ref[...]
    dinv_ref[...] = d64
    out0_ref[...] = o
    src_ref[...] = d64 * o


_mm0_call = pl.pallas_call(
    _mm0_body,
    grid=(NB,),
    in_specs=[
        pl.BlockSpec((NC, BLK, 16), lambda i: (0, i, 0)),
        pl.BlockSpec((BLK, D), lambda i: (i, 0)),
        pl.BlockSpec((D, H), lambda i: (0, 0)),
        pl.BlockSpec((1, H), lambda i: (0, 0)),
    ],
    out_specs=[
        pl.BlockSpec((BLK, H), lambda i: (i, 0)),
        pl.BlockSpec((BLK, H), lambda i: (i, 0)),
        pl.BlockSpec((BLK, H), lambda i: (i, 0)),
    ],
    out_shape=[
        jax.ShapeDtypeStruct((N2, H), jnp.float32),
        jax.ShapeDtypeStruct((N2, H), jnp.float32),
        jax.ShapeDtypeStruct((N2, H), jnp.float32),
    ],
)


def _ew_b_body(out0_ref, sp_ref, dinv_ref, a_ref, out1_ref, src_ref):
    d = dinv_ref[...]
    svm = sp_ref[0] + sp_ref[1]
    o1 = out0_ref[...] + a_ref[0, 0] * d * svm
    out1_ref[...] = o1
    src_ref[...] = d * o1


_ew_b_call = pl.pallas_call(
    _ew_b_body,
    grid=(NB,),
    in_specs=[
        pl.BlockSpec((BLK, H), lambda i: (i, 0)),
        pl.BlockSpec((NC, BLK, H), lambda i: (0, i, 0)),
        pl.BlockSpec((BLK, H), lambda i: (i, 0)),
        pl.BlockSpec(memory_space=pltpu.SMEM),
    ],
    out_specs=[
        pl.BlockSpec((BLK, H), lambda i: (i, 0)),
        pl.BlockSpec((BLK, H), lambda i: (i, 0)),
    ],
    out_shape=[
        jax.ShapeDtypeStruct((N2, H), jnp.float32),
        jax.ShapeDtypeStruct((N2, H), jnp.float32),
    ],
)


def _ew_c_body(sp_ref, dinv_ref, src_ref):
    d = dinv_ref[...]
    src_ref[...] = d * d * (sp_ref[0] + sp_ref[1])


_ew_c_call = pl.pallas_call(
    _ew_c_body,
    grid=(NB,),
    in_specs=[
        pl.BlockSpec((NC, BLK, H), lambda i: (0, i, 0)),
        pl.BlockSpec((BLK, H), lambda i: (i, 0)),
    ],
    out_specs=pl.BlockSpec((BLK, H), lambda i: (i, 0)),
    out_shape=jax.ShapeDtypeStruct((N2, H), jnp.float32),
)


def _relumm_body(out1_ref, sp_ref, dinv_ref, a_ref, w_ref, b_ref,
                 out0_ref, src_ref):
    d = dinv_ref[...]
    h = jnp.maximum(out1_ref[...] + a_ref[0, 0] * d * (sp_ref[0] + sp_ref[1]),
                    0.0)
    o = jnp.dot(h, w_ref[...], preferred_element_type=jnp.float32)
    o = o + b_ref[...]
    d64 = d
    out0_ref[...] = o
    src_ref[...] = d64 * o


_relumm_call = pl.pallas_call(
    _relumm_body,
    grid=(NB,),
    in_specs=[
        pl.BlockSpec((BLK, H), lambda i: (i, 0)),
        pl.BlockSpec((NC, BLK, H), lambda i: (0, i, 0)),
        pl.BlockSpec((BLK, H), lambda i: (i, 0)),
        pl.BlockSpec(memory_space=pltpu.SMEM),
        pl.BlockSpec((H, H), lambda i: (0, 0)),
        pl.BlockSpec((1, H), lambda i: (0, 0)),
    ],
    out_specs=[
        pl.BlockSpec((BLK, H), lambda i: (i, 0)),
        pl.BlockSpec((BLK, H), lambda i: (i, 0)),
    ],
    out_shape=[
        jax.ShapeDtypeStruct((N2, H), jnp.float32),
        jax.ShapeDtypeStruct((N2, H), jnp.float32),
    ],
)


def _final_body(out1_ref, sp_ref, dinv_ref, a_ref, bt_ref, wc_ref, bc_ref,
                out_ref, sums, counts):
    i = pl.program_id(0)

    @pl.when(i == 0)
    def _():
        sums[...] = jnp.zeros_like(sums)
        counts[...] = jnp.zeros_like(counts)

    d = dinv_ref[...]
    h = jnp.maximum(out1_ref[...] + a_ref[0, 0] * d * (sp_ref[0] + sp_ref[1]),
                    0.0)
    bt = bt_ref[0, 0]
    gids = lax.broadcasted_iota(jnp.int32, (G, BLK), 0)
    p = (bt[None, :] == gids).astype(jnp.float32)
    sums[...] += jnp.dot(p, h, preferred_element_type=jnp.float32)
    counts[...] += jnp.dot(p, jnp.ones((BLK, H), jnp.float32),
                           preferred_element_type=jnp.float32)

    @pl.when(i == NB - 1)
    def _():
        pooled = sums[...] / jnp.maximum(counts[...], 1.0)
        out_ref[...] = jnp.dot(pooled, wc_ref[...],
                               preferred_element_type=jnp.float32) + bc_ref[...]


_final_call = pl.pallas_call(
    _final_body,
    grid=(NB,),
    in_specs=[
        pl.BlockSpec((BLK, H), lambda i: (i, 0)),
        pl.BlockSpec((NC, BLK, H), lambda i: (0, i, 0)),
        pl.BlockSpec((BLK, H), lambda i: (i, 0)),
        pl.BlockSpec(memory_space=pltpu.SMEM),
        pl.BlockSpec((1, 1, BLK), lambda i: (i, 0, 0)),
        pl.BlockSpec((H, C), lambda i: (0, 0)),
        pl.BlockSpec((1, C), lambda i: (0, 0)),
    ],
    out_specs=pl.BlockSpec((G, C), lambda i: (0, 0)),
    out_shape=jax.ShapeDtypeStruct((G, C), jnp.float32),
    scratch_shapes=[
        pltpu.VMEM((G, H), jnp.float32),
        pltpu.VMEM((G, H), jnp.float32),
    ],
)


def kernel(x, edge_index, batch, W0, b0, a0, W1, b1, a1, W2, b2, a2, Wc, bc):
    row = edge_index[0]
    col = edge_index[1]
    pad = E_PAD - E
    rowp = jnp.concatenate(
        [row, jnp.full((pad,), N, jnp.int32)]).reshape(NW, CH, CHUNK)
    colp = jnp.concatenate(
        [col, jnp.full((pad,), N, jnp.int32)]).reshape(NW, CH, CHUNK)
    xp = jnp.pad(x, ((0, N2 - N), (0, 0)))
    batch2 = jnp.concatenate(
        [batch, jnp.full((N2 - N,), G, jnp.int32)]).reshape(NB, 1, BLK)
    zeros64 = jnp.zeros((N2, H), jnp.float32)
    zeros16 = jnp.zeros((N2, 16), jnp.float32)
    ones16 = jnp.ones((CHUNK, 16), jnp.float32)

    deg_part = _deg_call(rowp, ones16, zeros16)
    dinv64, out0, src = _mm0_call(deg_part, xp, W0.T, b0.reshape(1, H))

    out = None
    for (W, b, a) in ((W0, b0, a0), (W1, b1, a1), (W2, b2, a2)):
        a0s = a[0:1].reshape(1, 1)
        a1s = a[1:2].reshape(1, 1)
        sp = _prop_call(src, colp, rowp, zeros64)
        out1, src = _ew_b_call(out0, sp, dinv64, a0s)
        sp = _prop_call(src, colp, rowp, zeros64)
        sp = _prop3_call(sp, dinv64, colp, rowp, zeros64)
        if W is W2:
            out = _final_call(out1, sp, dinv64, a1s, batch2, Wc.T,
                              bc.reshape(1, C))
        elif W is W0:
            out0, src = _relumm_call(out1, sp, dinv64, a1s, W1.T,
                                     b1.reshape(1, H))
        else:
            out0, src = _relumm_call(out1, sp, dinv64, a1s, W2.T,
                                     b2.reshape(1, H))
    return out
